# Initial kernel scaffold; baseline (speedup 1.0000x reference)
import jax
import jax.numpy as jnp
from jax.experimental import pallas as pl


def kernel(x, batch):
    raise NotImplementedError("write your pallas kernel here")


# speedup vs baseline: 3.1720x; 3.1720x over previous
import functools

import jax
import jax.numpy as jnp
from jax import lax
from jax.experimental import pallas as pl
from jax.experimental.pallas import tpu as pltpu
from jax.experimental.pallas import tpu_sc as plsc

NUM_SEG = 512
D_FEAT = 128
N_ROWS = 100000
NUM_CORES = 2
NUM_TILES = 16
NUM_WORKERS = NUM_CORES * NUM_TILES
CHUNK = 125
CH_PAD = 128
NUM_CHUNKS = N_ROWS // CHUNK
CHUNKS_PER_W = NUM_CHUNKS // NUM_WORKERS

_mesh = plsc.VectorSubcoreMesh(core_axis_name="c", subcore_axis_name="s")


@functools.partial(
    pl.kernel,
    out_type=(
        jax.ShapeDtypeStruct((NUM_CORES * NUM_SEG, D_FEAT), jnp.float32),
        jax.ShapeDtypeStruct((NUM_CORES * NUM_SEG, 1), jnp.float32),
    ),
    mesh=_mesh,
    compiler_params=pltpu.CompilerParams(use_tc_tiling_on_sc=False),
    scratch_types=[
        pltpu.VMEM((CH_PAD, D_FEAT), jnp.float32),
        pltpu.VMEM((CH_PAD,), jnp.int32),
        pltpu.VMEM((CH_PAD, 1), jnp.float32),
        pltpu.VMEM_SHARED((NUM_SEG, D_FEAT), jnp.float32),
        pltpu.VMEM_SHARED((NUM_SEG, 1), jnp.float32),
    ],
)
def _sc_segment_sum(x_hbm, b2_hbm, ones_hbm, zf_hbm, zc_hbm,
                    psums_hbm, pcnts_hbm, rows, idx, ones, acc, cnt):
    cid = lax.axis_index("c")
    sid = lax.axis_index("s")
    w = cid * NUM_TILES + sid
    rows_per_tile = NUM_SEG // NUM_TILES
    seg_base = sid * rows_per_tile

    pltpu.sync_copy(ones_hbm, ones)
    pltpu.sync_copy(zf_hbm.at[pl.ds(0, CH_PAD - CHUNK)],
                    rows.at[pl.ds(CHUNK, CH_PAD - CHUNK)])
    pltpu.sync_copy(zf_hbm.at[pl.ds(seg_base, rows_per_tile)],
                    acc.at[pl.ds(seg_base, rows_per_tile)])
    pltpu.sync_copy(zc_hbm.at[pl.ds(seg_base, rows_per_tile)],
                    cnt.at[pl.ds(seg_base, rows_per_tile)])
    plsc.subcore_barrier()

    def body(c, carry):
        j = w * CHUNKS_PER_W + c
        pltpu.sync_copy(b2_hbm.at[j], idx)
        pltpu.sync_copy(x_hbm.at[pl.ds(j * CHUNK, CHUNK)],
                        rows.at[pl.ds(0, CHUNK)])
        pltpu.sync_copy(rows, acc.at[idx], add=True)
        pltpu.sync_copy(ones, cnt.at[idx], add=True)
        return carry

    lax.fori_loop(0, CHUNKS_PER_W, body, 0)
    plsc.subcore_barrier()

    out_base = cid * NUM_SEG + seg_base
    pltpu.sync_copy(acc.at[pl.ds(seg_base, rows_per_tile)],
                    psums_hbm.at[pl.ds(out_base, rows_per_tile)])
    pltpu.sync_copy(cnt.at[pl.ds(seg_base, rows_per_tile)],
                    pcnts_hbm.at[pl.ds(out_base, rows_per_tile)])


def _combine_body(ps_ref, pc_ref, o_ref):
    s = ps_ref[0] + ps_ref[1]
    c = pc_ref[0] + pc_ref[1]
    scale = 1.0 / (jnp.maximum(c, 1.0) * jnp.sqrt(c + 1e-6))
    o_ref[...] = s * scale[:, None]


def kernel(x, batch):
    b2 = jnp.pad(batch.reshape(NUM_CHUNKS, CHUNK),
                 ((0, 0), (0, CH_PAD - CHUNK)))
    ones_col = (lax.broadcasted_iota(jnp.int32, (CH_PAD, 1), 0)
                < CHUNK).astype(jnp.float32)
    zf = jnp.zeros((NUM_SEG, D_FEAT), jnp.float32)
    zc = jnp.zeros((NUM_SEG, 1), jnp.float32)
    psums, pcnts = _sc_segment_sum(x, b2, ones_col, zf, zc)
    protein_repr = pl.pallas_call(
        _combine_body,
        out_shape=jax.ShapeDtypeStruct((NUM_SEG, D_FEAT), jnp.float32),
    )(psums.reshape(NUM_CORES, NUM_SEG, D_FEAT),
      pcnts.reshape(NUM_CORES, NUM_SEG))
    return x, protein_repr


# speedup vs baseline: 3.7155x; 1.1713x over previous
import functools

import jax
import jax.numpy as jnp
from jax import lax
from jax.experimental import pallas as pl
from jax.experimental.pallas import tpu as pltpu
from jax.experimental.pallas import tpu_sc as plsc

NUM_SEG = 512
D_FEAT = 128
N_ROWS = 100000
NUM_CORES = 2
NUM_TILES = 16
NUM_WORKERS = NUM_CORES * NUM_TILES
CHUNK = 125
CH_PAD = 128
NUM_CHUNKS = N_ROWS // CHUNK
CHUNKS_PER_W = NUM_CHUNKS // NUM_WORKERS
NUM_PAIRS = CHUNKS_PER_W // 2

_mesh = plsc.VectorSubcoreMesh(core_axis_name="c", subcore_axis_name="s")


@functools.partial(
    pl.kernel,
    out_type=(
        jax.ShapeDtypeStruct((NUM_CORES * NUM_SEG, D_FEAT), jnp.float32),
        jax.ShapeDtypeStruct((NUM_CORES * NUM_SEG, 1), jnp.float32),
    ),
    mesh=_mesh,
    compiler_params=pltpu.CompilerParams(use_tc_tiling_on_sc=False),
    scratch_types=[
        pltpu.VMEM((CH_PAD, D_FEAT), jnp.float32),
        pltpu.VMEM((CH_PAD, D_FEAT), jnp.float32),
        pltpu.VMEM((CH_PAD,), jnp.int32),
        pltpu.VMEM((CH_PAD,), jnp.int32),
        pltpu.VMEM((CH_PAD, 1), jnp.float32),
        pltpu.VMEM_SHARED((NUM_SEG, D_FEAT), jnp.float32),
        pltpu.VMEM_SHARED((NUM_SEG, 1), jnp.float32),
        pltpu.SemaphoreType.DMA,
        pltpu.SemaphoreType.DMA,
    ],
)
def _sc_segment_sum(x_hbm, b2_hbm, ones_hbm, zf_hbm, zc_hbm,
                    psums_hbm, pcnts_hbm, rows_a, rows_b, idx_a, idx_b,
                    ones, acc, cnt, lsem_a, lsem_b):
    cid = lax.axis_index("c")
    sid = lax.axis_index("s")
    w = cid * NUM_TILES + sid
    rows_per_tile = NUM_SEG // NUM_TILES
    seg_base = sid * rows_per_tile

    pltpu.sync_copy(ones_hbm, ones)
    pltpu.sync_copy(zf_hbm.at[pl.ds(0, CH_PAD - CHUNK)],
                    rows_a.at[pl.ds(CHUNK, CH_PAD - CHUNK)])
    pltpu.sync_copy(zf_hbm.at[pl.ds(0, CH_PAD - CHUNK)],
                    rows_b.at[pl.ds(CHUNK, CH_PAD - CHUNK)])
    pltpu.sync_copy(zf_hbm.at[pl.ds(seg_base, rows_per_tile)],
                    acc.at[pl.ds(seg_base, rows_per_tile)])
    pltpu.sync_copy(zc_hbm.at[pl.ds(seg_base, rows_per_tile)],
                    cnt.at[pl.ds(seg_base, rows_per_tile)])
    plsc.subcore_barrier()

    def load(c, buf, sem):
        return pltpu.async_copy(
            x_hbm.at[pl.ds((w * CHUNKS_PER_W + c) * CHUNK, CHUNK)],
            buf.at[pl.ds(0, CHUNK)], sem)

    def scatter(c, buf, idx):
        pltpu.sync_copy(b2_hbm.at[w * CHUNKS_PER_W + c], idx)
        pltpu.sync_copy(buf, acc.at[idx], add=True)
        pltpu.sync_copy(ones, cnt.at[idx], add=True)

    load(0, rows_a, lsem_a).wait()

    def pair(t, carry):
        c0 = 2 * t
        lb = load(c0 + 1, rows_b, lsem_b)
        scatter(c0, rows_a, idx_a)
        lb.wait()
        la = load(c0 + 2, rows_a, lsem_a)
        scatter(c0 + 1, rows_b, idx_b)
        la.wait()
        return carry

    lax.fori_loop(0, NUM_PAIRS, pair, 0)
    scatter(CHUNKS_PER_W - 1, rows_a, idx_a)
    plsc.subcore_barrier()

    out_base = cid * NUM_SEG + seg_base
    pltpu.sync_copy(acc.at[pl.ds(seg_base, rows_per_tile)],
                    psums_hbm.at[pl.ds(out_base, rows_per_tile)])
    pltpu.sync_copy(cnt.at[pl.ds(seg_base, rows_per_tile)],
                    pcnts_hbm.at[pl.ds(out_base, rows_per_tile)])


def _combine_body(ps_ref, pc_ref, o_ref):
    s = ps_ref[0] + ps_ref[1]
    c = pc_ref[0] + pc_ref[1]
    scale = 1.0 / (jnp.maximum(c, 1.0) * jnp.sqrt(c + 1e-6))
    o_ref[...] = s * scale[:, None]


def kernel(x, batch):
    b2 = jnp.pad(batch.reshape(NUM_CHUNKS, CHUNK),
                 ((0, 0), (0, CH_PAD - CHUNK)))
    ones_col = (lax.broadcasted_iota(jnp.int32, (CH_PAD, 1), 0)
                < CHUNK).astype(jnp.float32)
    zf = jnp.zeros((NUM_SEG, D_FEAT), jnp.float32)
    zc = jnp.zeros((NUM_SEG, 1), jnp.float32)
    psums, pcnts = _sc_segment_sum(x, b2, ones_col, zf, zc)
    protein_repr = pl.pallas_call(
        _combine_body,
        out_shape=jax.ShapeDtypeStruct((NUM_SEG, D_FEAT), jnp.float32),
    )(psums.reshape(NUM_CORES, NUM_SEG, D_FEAT),
      pcnts.reshape(NUM_CORES, NUM_SEG))
    return x, protein_repr


# speedup vs baseline: 3.8772x; 1.0435x over previous
import functools

import jax
import jax.numpy as jnp
from jax import lax
from jax.experimental import pallas as pl
from jax.experimental.pallas import tpu as pltpu
from jax.experimental.pallas import tpu_sc as plsc

NUM_SEG = 512
D_FEAT = 128
N_ROWS = 100000
NUM_CORES = 2
NUM_TILES = 16
NUM_WORKERS = NUM_CORES * NUM_TILES
CHUNK = 125
CH_PAD = 128
NUM_CHUNKS = N_ROWS // CHUNK
CHUNKS_PER_W = NUM_CHUNKS // NUM_WORKERS
NUM_PAIRS = CHUNKS_PER_W // 2

_mesh = plsc.VectorSubcoreMesh(core_axis_name="c", subcore_axis_name="s")


@functools.partial(
    pl.kernel,
    out_type=(
        jax.ShapeDtypeStruct((NUM_CORES * NUM_SEG, D_FEAT), jnp.float32),
        jax.ShapeDtypeStruct((NUM_CORES * NUM_SEG, 1), jnp.float32),
    ),
    mesh=_mesh,
    compiler_params=pltpu.CompilerParams(use_tc_tiling_on_sc=False),
    scratch_types=[
        pltpu.VMEM((CH_PAD, D_FEAT), jnp.float32),
        pltpu.VMEM((CH_PAD, D_FEAT), jnp.float32),
        pltpu.VMEM((CH_PAD,), jnp.int32),
        pltpu.VMEM((CH_PAD,), jnp.int32),
        pltpu.VMEM((CH_PAD, 1), jnp.float32),
        pltpu.VMEM_SHARED((NUM_SEG, D_FEAT), jnp.float32),
        pltpu.VMEM_SHARED((NUM_SEG, 1), jnp.float32),
        pltpu.SemaphoreType.DMA,
        pltpu.SemaphoreType.DMA,
    ],
)
def _sc_segment_sum(x_hbm, b2_hbm, ones_hbm, zf_hbm, zc_hbm,
                    psums_hbm, pcnts_hbm, rows_a, rows_b, idx_a, idx_b,
                    ones, acc, cnt, lsem_a, lsem_b):
    cid = lax.axis_index("c")
    sid = lax.axis_index("s")
    w = cid * NUM_TILES + sid
    rows_per_tile = NUM_SEG // NUM_TILES
    seg_base = sid * rows_per_tile

    pltpu.sync_copy(ones_hbm, ones)
    pltpu.sync_copy(zf_hbm.at[pl.ds(0, CH_PAD - CHUNK)],
                    rows_a.at[pl.ds(CHUNK, CH_PAD - CHUNK)])
    pltpu.sync_copy(zf_hbm.at[pl.ds(0, CH_PAD - CHUNK)],
                    rows_b.at[pl.ds(CHUNK, CH_PAD - CHUNK)])
    pltpu.sync_copy(zf_hbm.at[pl.ds(seg_base, rows_per_tile)],
                    acc.at[pl.ds(seg_base, rows_per_tile)])
    pltpu.sync_copy(zc_hbm.at[pl.ds(seg_base, rows_per_tile)],
                    cnt.at[pl.ds(seg_base, rows_per_tile)])
    plsc.subcore_barrier()

    def load(c, buf, idx, sem):
        d0 = pltpu.async_copy(
            x_hbm.at[pl.ds((w * CHUNKS_PER_W + c) * CHUNK, CHUNK)],
            buf.at[pl.ds(0, CHUNK)], sem)
        d1 = pltpu.async_copy(b2_hbm.at[w * CHUNKS_PER_W + c], idx, sem)
        return d0, d1

    def wait(d):
        d[0].wait()
        d[1].wait()

    def scatter(buf, idx):
        pltpu.sync_copy(buf, acc.at[idx], add=True)
        pltpu.sync_copy(ones, cnt.at[idx], add=True)

    wait(load(0, rows_a, idx_a, lsem_a))

    def pair(t, carry):
        c0 = 2 * t
        lb = load(c0 + 1, rows_b, idx_b, lsem_b)
        scatter(rows_a, idx_a)
        wait(lb)
        la = load(c0 + 2, rows_a, idx_a, lsem_a)
        scatter(rows_b, idx_b)
        wait(la)
        return carry

    lax.fori_loop(0, NUM_PAIRS, pair, 0)
    scatter(rows_a, idx_a)
    plsc.subcore_barrier()

    out_base = cid * NUM_SEG + seg_base
    pltpu.sync_copy(acc.at[pl.ds(seg_base, rows_per_tile)],
                    psums_hbm.at[pl.ds(out_base, rows_per_tile)])
    pltpu.sync_copy(cnt.at[pl.ds(seg_base, rows_per_tile)],
                    pcnts_hbm.at[pl.ds(out_base, rows_per_tile)])


def _combine_body(ps_ref, pc_ref, o_ref):
    s = ps_ref[0] + ps_ref[1]
    c = pc_ref[0] + pc_ref[1]
    scale = 1.0 / (jnp.maximum(c, 1.0) * jnp.sqrt(c + 1e-6))
    o_ref[...] = s * scale[:, None]


def kernel(x, batch):
    b2 = jnp.pad(batch.reshape(NUM_CHUNKS, CHUNK),
                 ((0, 0), (0, CH_PAD - CHUNK)))
    ones_col = (lax.broadcasted_iota(jnp.int32, (CH_PAD, 1), 0)
                < CHUNK).astype(jnp.float32)
    zf = jnp.zeros((NUM_SEG, D_FEAT), jnp.float32)
    zc = jnp.zeros((NUM_SEG, 1), jnp.float32)
    psums, pcnts = _sc_segment_sum(x, b2, ones_col, zf, zc)
    protein_repr = pl.pallas_call(
        _combine_body,
        out_shape=jax.ShapeDtypeStruct((NUM_SEG, D_FEAT), jnp.float32),
    )(psums.reshape(NUM_CORES, NUM_SEG, D_FEAT),
      pcnts.reshape(NUM_CORES, NUM_SEG))
    return x, protein_repr


# speedup vs baseline: 3.9939x; 1.0301x over previous
import functools

import jax
import jax.numpy as jnp
from jax import lax
from jax.experimental import pallas as pl
from jax.experimental.pallas import tpu as pltpu
from jax.experimental.pallas import tpu_sc as plsc

NUM_SEG = 512
D_FEAT = 128
N_ROWS = 100000
NUM_CORES = 2
NUM_TILES = 16
NUM_WORKERS = NUM_CORES * NUM_TILES
CHUNK = 125
CH_PAD = 128
NUM_CHUNKS = N_ROWS // CHUNK
CHUNKS_PER_W = NUM_CHUNKS // NUM_WORKERS
NUM_PAIRS = CHUNKS_PER_W // 2
LANES = 16

_mesh = plsc.VectorSubcoreMesh(core_axis_name="c", subcore_axis_name="s")


@functools.partial(
    pl.kernel,
    out_type=(
        jax.ShapeDtypeStruct((NUM_CORES * NUM_SEG, D_FEAT), jnp.float32),
        jax.ShapeDtypeStruct((NUM_WORKERS, NUM_SEG), jnp.float32),
    ),
    mesh=_mesh,
    compiler_params=pltpu.CompilerParams(use_tc_tiling_on_sc=False,
                                         needs_layout_passes=False),
    scratch_types=[
        pltpu.VMEM((CH_PAD, D_FEAT), jnp.float32),
        pltpu.VMEM((CH_PAD, D_FEAT), jnp.float32),
        pltpu.VMEM((CH_PAD,), jnp.int32),
        pltpu.VMEM((CH_PAD,), jnp.int32),
        pltpu.VMEM((NUM_SEG,), jnp.float32),
        pltpu.VMEM_SHARED((NUM_SEG, D_FEAT), jnp.float32),
        pltpu.SemaphoreType.DMA,
        pltpu.SemaphoreType.DMA,
        pltpu.SemaphoreType.DMA,
        pltpu.SemaphoreType.DMA,
    ],
)
def _sc_segment_sum(x_hbm, b2_hbm, zf_hbm, psums_hbm, pcnts_hbm,
                    rows_a, rows_b, idx_a, idx_b, cnts, acc,
                    lsem_a, lsem_b, ssem_a, ssem_b):
    cid = lax.axis_index("c")
    sid = lax.axis_index("s")
    w = cid * NUM_TILES + sid
    rows_per_tile = NUM_SEG // NUM_TILES
    seg_base = sid * rows_per_tile

    pltpu.sync_copy(zf_hbm.at[pl.ds(0, CH_PAD - CHUNK)],
                    rows_a.at[pl.ds(CHUNK, CH_PAD - CHUNK)])
    pltpu.sync_copy(zf_hbm.at[pl.ds(0, CH_PAD - CHUNK)],
                    rows_b.at[pl.ds(CHUNK, CH_PAD - CHUNK)])
    pltpu.sync_copy(zf_hbm.at[pl.ds(seg_base, rows_per_tile)],
                    acc.at[pl.ds(seg_base, rows_per_tile)])
    zero16 = jnp.zeros((LANES,), jnp.float32)

    def zbody(r, carry):
        cnts[pl.ds(r * LANES, LANES)] = zero16
        return carry

    lax.fori_loop(0, NUM_SEG // LANES, zbody, 0)
    plsc.subcore_barrier()

    one16 = jnp.ones((LANES,), jnp.float32)
    tail_mask = lax.iota(jnp.int32, LANES) < (CHUNK % LANES)

    def load(c, buf, idx, sem):
        d0 = pltpu.async_copy(
            x_hbm.at[pl.ds((w * CHUNKS_PER_W + c) * CHUNK, CHUNK)],
            buf.at[pl.ds(0, CHUNK)], sem)
        d1 = pltpu.async_copy(b2_hbm.at[w * CHUNKS_PER_W + c], idx, sem)
        return d0, d1

    def wait(d):
        d[0].wait()
        d[1].wait()

    def scatter(buf, idx, sem):
        return pltpu.async_copy(buf, acc.at[idx], sem, add=True)

    def count(idx):
        for k in range(CHUNK // LANES):
            seg = idx[pl.ds(k * LANES, LANES)]
            plsc.addupdate_scatter(cnts, [seg], one16)
        seg = idx[pl.ds((CHUNK // LANES) * LANES, LANES)]
        plsc.addupdate_scatter(cnts, [seg], one16, mask=tail_mask)

    wait(load(0, rows_a, idx_a, lsem_a))

    def pair(t, carry):
        c0 = 2 * t
        lb = load(c0 + 1, rows_b, idx_b, lsem_b)
        sa = scatter(rows_a, idx_a, ssem_a)
        count(idx_a)
        wait(lb)
        sb = scatter(rows_b, idx_b, ssem_b)
        count(idx_b)
        sa.wait()
        la = load(c0 + 2, rows_a, idx_a, lsem_a)
        sb.wait()
        wait(la)
        return carry

    lax.fori_loop(0, NUM_PAIRS, pair, 0)

    st = scatter(rows_a, idx_a, ssem_a)
    count(idx_a)
    st.wait()
    plsc.subcore_barrier()

    out_base = cid * NUM_SEG + seg_base
    pltpu.sync_copy(acc.at[pl.ds(seg_base, rows_per_tile)],
                    psums_hbm.at[pl.ds(out_base, rows_per_tile)])
    pltpu.sync_copy(cnts, pcnts_hbm.at[w])


def _combine_body(ps_ref, pc_ref, o_ref):
    s = ps_ref[0] + ps_ref[1]
    c = jnp.sum(pc_ref[...], axis=0)
    scale = 1.0 / (jnp.maximum(c, 1.0) * jnp.sqrt(c + 1e-6))
    o_ref[...] = s * scale[:, None]


def kernel(x, batch):
    b2 = jnp.pad(batch.reshape(NUM_CHUNKS, CHUNK),
                 ((0, 0), (0, CH_PAD - CHUNK)))
    zf = jnp.zeros((NUM_SEG, D_FEAT), jnp.float32)
    psums, pcnts = _sc_segment_sum(x, b2, zf)
    protein_repr = pl.pallas_call(
        _combine_body,
        out_shape=jax.ShapeDtypeStruct((NUM_SEG, D_FEAT), jnp.float32),
    )(psums.reshape(NUM_CORES, NUM_SEG, D_FEAT), pcnts)
    return x, protein_repr
